# baseline (speedup 1.0000x reference)
import functools

import jax
import jax.numpy as jnp
from jax import lax
from jax.experimental import pallas as pl
from jax.experimental.pallas import tpu as pltpu
from jax.experimental.pallas import tpu_sc as plsc

N_NODES = 10000
N_EDGES = 320000
D = 128
OUT = 64
NC = 2
NS = 16
NW = NC * NS
EPW = N_EDGES // NW
LANES = 16
CHUNKS = EPW // LANES
WIN = ((EPW // 128) + 1) * 128
WCHUNKS = WIN // LANES
BLK = 8
NBLK = WCHUNKS // BLK
NOPOS = 1 << 27


def _sc_filter_gather(emb_hbm, edges_hbm, src2d_hbm, sum_out, deg_out,
                      edgeb, srcb, srcg, blkb, idxb, rowsb, accb, degb,
                      degfb, sem_w, sem_c, sem_s, sem_g):
    wid = lax.axis_index("s") * NC + lax.axis_index("c")
    base = wid * EPW
    ab = (base // 128) * 128
    lo = (base - ab) // LANES
    pltpu.async_copy(edges_hbm.at[1, pl.ds(ab, WIN)], edgeb, sem_w).wait()

    ones = jnp.ones((LANES,), jnp.int32)

    def neutralize(c, carry):
        edgeb[pl.ds(c * LANES, LANES)] = ones
        return carry

    lax.fori_loop(0, lo, neutralize, 0)
    lax.fori_loop(lo + CHUNKS, WCHUNKS, neutralize, 0)

    zf = jnp.zeros((LANES,), jnp.float32)
    for k in range(D // LANES):
        accb[pl.ds(k * LANES, LANES)] = zf
    degb[...] = jnp.zeros((LANES,), jnp.int32)

    def acc_row(r, c2):
        for k in range(D // LANES):
            sl = pl.ds(k * LANES, LANES)
            accb[sl] = accb[sl] + rowsb[r, sl]
        return c2

    def screen_blk(b, carry):
        cv, pmin = carry
        off0 = b * BLK * LANES
        for t in range(BLK):
            c = b * BLK + t
            dv = edgeb[pl.ds(off0 + t * LANES, LANES)]
            m = dv == 0
            cv = cv + jnp.where(m, 1, 0).astype(jnp.int32)
            pmin = jnp.minimum(pmin, jnp.where(m, c, NOPOS))
        return cv, pmin

    cv0 = jnp.zeros((LANES,), jnp.int32)
    pm0 = jnp.full((LANES,), NOPOS, jnp.int32)
    cv, pmin = lax.fori_loop(0, NBLK, screen_blk, (cv0, pm0))

    def fine(c, carry):
        off = c * LANES
        dv = edgeb[pl.ds(off, LANES)]
        m = dv == 0
        mi = jnp.where(m, 1, 0).astype(jnp.int32)
        cnt = jnp.sum(mi)

        @pl.when(cnt > 0)
        def _():
            j = c % BLK
            sv = srcb[pl.ds(j * LANES, LANES)]
            idxb[...] = jnp.zeros((LANES,), jnp.int32)
            plsc.store_compressed(idxb.at[pl.ds(0, LANES)], sv, mask=m)
            degb[...] = degb[...] + mi
            pltpu.async_copy(emb_hbm.at[idxb], rowsb, sem_g).wait()
            lax.fori_loop(0, cnt, acc_row, 0)

        return carry

    def rescan(b, carry):
        off0 = b * BLK * LANES
        mn = edgeb[pl.ds(off0, LANES)]
        for t in range(1, BLK):
            mn = jnp.minimum(mn, edgeb[pl.ds(off0 + t * LANES, LANES)])

        @pl.when(jnp.min(mn) == 0)
        def _():
            pltpu.async_copy(
                edges_hbm.at[0, pl.ds(ab + off0, BLK * LANES)],
                srcb, sem_s).wait()
            lax.fori_loop(b * BLK, (b + 1) * BLK, fine, 0)

        return carry

    total = jnp.sum(cv)

    @pl.when(total > 0)
    def _():
        mx = jnp.max(cv)

        @pl.when(mx == 1)
        def _():
            iot = lax.iota(jnp.int32, 16)
            hasm = cv == 1
            pm = jnp.where(hasm, pmin, 0)
            gpos = ab + pm * LANES + iot
            blkb[...] = jnp.right_shift(gpos, 7)
            colv = jnp.bitwise_and(gpos, 127)
            pltpu.async_copy(src2d_hbm.at[blkb], srcg, sem_c).wait()
            srcv = plsc.load_gather(srcg, [iot, colv])
            srcm = jnp.where(hasm, srcv, 0)
            idxb[...] = jnp.zeros((LANES,), jnp.int32)
            plsc.store_compressed(idxb.at[pl.ds(0, LANES)], srcm, mask=hasm)
            degb[...] = cv
            pltpu.async_copy(emb_hbm.at[idxb], rowsb, sem_g).wait()
            lax.fori_loop(0, total, acc_row, 0)

        @pl.when(mx > 1)
        def _():
            lax.fori_loop(0, NBLK, rescan, 0)

    pltpu.sync_copy(accb, sum_out.at[wid])
    dt = jnp.sum(degb[...]).astype(jnp.float32)
    for k in range(D // LANES):
        degfb[pl.ds(k * LANES, LANES)] = jnp.full((LANES,), dt, jnp.float32)
    pltpu.sync_copy(degfb, deg_out.at[wid])


def _tc_finish(part_ref, deg_ref, emb_ref, ws_ref, wn_ref, bs_ref,
               wc_ref, bc_ref, out_ref):
    s = jnp.sum(part_ref[...], axis=0, keepdims=True)
    deg = jnp.sum(deg_ref[...], axis=0, keepdims=True)[0:1, 0:1]
    agg = s / jnp.maximum(deg, 1.0)
    e0 = emb_ref[0:1, :]
    h = jnp.maximum(
        jnp.dot(e0, ws_ref[...], preferred_element_type=jnp.float32)
        + jnp.dot(agg, wn_ref[...], preferred_element_type=jnp.float32)
        + bs_ref[...][None, :], 0.0)
    out_ref[...] = (jnp.dot(h, wc_ref[...], preferred_element_type=jnp.float32)
                    + bc_ref[...][None, :])


def kernel(embedding, edges, W_self, W_neigh, b_sage, W_cls, b_cls):
    edges = edges.astype(jnp.int32)
    src2d = edges[0].reshape(N_EDGES // 128, 128)

    mesh = plsc.VectorSubcoreMesh(core_axis_name="c", subcore_axis_name="s")
    sc_call = functools.partial(
        pl.kernel,
        mesh=mesh,
        compiler_params=pltpu.CompilerParams(needs_layout_passes=False),
        out_type=(
            jax.ShapeDtypeStruct((NW, D), jnp.float32),
            jax.ShapeDtypeStruct((NW, D), jnp.float32),
        ),
        scratch_types=[
            pltpu.VMEM((WIN,), jnp.int32),
            pltpu.VMEM((BLK * LANES,), jnp.int32),
            pltpu.VMEM((LANES, 128), jnp.int32),
            pltpu.VMEM((LANES,), jnp.int32),
            pltpu.VMEM((LANES,), jnp.int32),
            pltpu.VMEM((LANES, D), jnp.float32),
            pltpu.VMEM((D,), jnp.float32),
            pltpu.VMEM((LANES,), jnp.int32),
            pltpu.VMEM((D,), jnp.float32),
            pltpu.SemaphoreType.DMA,
            pltpu.SemaphoreType.DMA,
            pltpu.SemaphoreType.DMA,
            pltpu.SemaphoreType.DMA,
        ],
    )
    partials, degs = sc_call(_sc_filter_gather)(embedding, edges, src2d)

    out = pl.pallas_call(
        _tc_finish,
        out_shape=jax.ShapeDtypeStruct((1, OUT), jnp.float32),
        grid=(1,),
        in_specs=[
            pl.BlockSpec((NW, D), lambda i: (0, 0)),
            pl.BlockSpec((NW, D), lambda i: (0, 0)),
            pl.BlockSpec((8, D), lambda i: (0, 0)),
            pl.BlockSpec((D, D), lambda i: (0, 0)),
            pl.BlockSpec((D, D), lambda i: (0, 0)),
            pl.BlockSpec((D,), lambda i: (0,)),
            pl.BlockSpec((D, OUT), lambda i: (0, 0)),
            pl.BlockSpec((OUT,), lambda i: (0,)),
        ],
        out_specs=pl.BlockSpec((1, OUT), lambda i: (0, 0)),
    )(partials, degs, embedding, W_self, W_neigh, b_sage, W_cls, b_cls)

    return out


# speedup vs baseline: 1.3862x; 1.3862x over previous
import functools

import jax
import jax.numpy as jnp
from jax import lax
from jax.experimental import pallas as pl
from jax.experimental.pallas import tpu as pltpu
from jax.experimental.pallas import tpu_sc as plsc

N_NODES = 10000
N_EDGES = 320000
D = 128
OUT = 64
NC = 2
NS = 16
NW = NC * NS
EPW = N_EDGES // NW
LANES = 16
CHUNKS = EPW // LANES
WIN = ((EPW // 128) + 1) * 128
WCHUNKS = WIN // LANES
BLK = 8
NBLK = WCHUNKS // BLK
NOPOS = 1 << 27


def _sc_filter_gather(emb_hbm, edges_hbm, sum_out, deg_out,
                      edgeb, srcwb, idxb, rowsb, accb, degb,
                      degfb, sem_w, sem_s, sem_g):
    wid = lax.axis_index("s") * NC + lax.axis_index("c")
    base = wid * EPW
    ab = (base // 128) * 128
    lo = (base - ab) // LANES
    dcp = pltpu.async_copy(edges_hbm.at[1, pl.ds(ab, WIN)], edgeb, sem_w)
    scp = pltpu.async_copy(edges_hbm.at[0, pl.ds(ab, WIN)], srcwb, sem_s)
    dcp.wait()

    ones = jnp.ones((LANES,), jnp.int32)

    def neutralize(c, carry):
        edgeb[pl.ds(c * LANES, LANES)] = ones
        return carry

    lax.fori_loop(0, lo, neutralize, 0)
    lax.fori_loop(lo + CHUNKS, WCHUNKS, neutralize, 0)

    zf = jnp.zeros((LANES,), jnp.float32)
    for k in range(D // LANES):
        accb[pl.ds(k * LANES, LANES)] = zf
    degb[...] = jnp.zeros((LANES,), jnp.int32)

    def acc_row(r, c2):
        for k in range(D // LANES):
            sl = pl.ds(k * LANES, LANES)
            accb[sl] = accb[sl] + rowsb[r, sl]
        return c2

    NACC = 4

    def screen_blk(b, carry):
        st = list(carry)
        off0 = b * BLK * LANES
        for t in range(BLK):
            c = b * BLK + t
            dv = edgeb[pl.ds(off0 + t * LANES, LANES)]
            m = dv == 0
            a = t % NACC
            st[2 * a] = st[2 * a] + jnp.where(m, 1, 0).astype(jnp.int32)
            st[2 * a + 1] = jnp.minimum(st[2 * a + 1],
                                        jnp.where(m, c, NOPOS))
        return tuple(st)

    cv0 = jnp.zeros((LANES,), jnp.int32)
    pm0 = jnp.full((LANES,), NOPOS, jnp.int32)
    st = lax.fori_loop(0, NBLK, screen_blk, (cv0, pm0) * NACC)
    cv = st[0] + st[2] + st[4] + st[6]
    pmin = jnp.minimum(jnp.minimum(st[1], st[3]),
                       jnp.minimum(st[5], st[7]))

    def fine(c, carry):
        off = c * LANES
        dv = edgeb[pl.ds(off, LANES)]
        m = dv == 0
        mi = jnp.where(m, 1, 0).astype(jnp.int32)
        cnt = jnp.sum(mi)

        @pl.when(cnt > 0)
        def _():
            sv = srcwb[pl.ds(off, LANES)]
            idxb[...] = jnp.zeros((LANES,), jnp.int32)
            plsc.store_compressed(idxb.at[pl.ds(0, LANES)], sv, mask=m)
            degb[...] = degb[...] + mi
            pltpu.async_copy(emb_hbm.at[idxb], rowsb, sem_g).wait()
            lax.fori_loop(0, cnt, acc_row, 0)

        return carry

    def rescan(b, carry):
        off0 = b * BLK * LANES
        mn = edgeb[pl.ds(off0, LANES)]
        for t in range(1, BLK):
            mn = jnp.minimum(mn, edgeb[pl.ds(off0 + t * LANES, LANES)])

        @pl.when(jnp.min(mn) == 0)
        def _():
            lax.fori_loop(b * BLK, (b + 1) * BLK, fine, 0)

        return carry

    total = jnp.sum(cv)
    scp.wait()

    @pl.when(total > 0)
    def _():
        mx = jnp.max(cv)

        @pl.when(mx == 1)
        def _():
            iot = lax.iota(jnp.int32, 16)
            hasm = cv == 1
            pm = jnp.where(hasm, pmin, 0)
            wpos = pm * LANES + iot
            srcv = plsc.load_gather(srcwb, [wpos])
            srcm = jnp.where(hasm, srcv, 0)
            idxb[...] = jnp.zeros((LANES,), jnp.int32)
            plsc.store_compressed(idxb.at[pl.ds(0, LANES)], srcm, mask=hasm)
            degb[...] = cv
            pltpu.async_copy(emb_hbm.at[idxb], rowsb, sem_g).wait()
            lax.fori_loop(0, total, acc_row, 0)

        @pl.when(mx > 1)
        def _():
            lax.fori_loop(0, NBLK, rescan, 0)

    pltpu.sync_copy(accb, sum_out.at[wid])
    dt = jnp.sum(degb[...]).astype(jnp.float32)
    for k in range(D // LANES):
        degfb[pl.ds(k * LANES, LANES)] = jnp.full((LANES,), dt, jnp.float32)
    pltpu.sync_copy(degfb, deg_out.at[wid])


def _tc_finish(part_ref, deg_ref, emb_ref, ws_ref, wn_ref, bs_ref,
               wc_ref, bc_ref, out_ref):
    s = jnp.sum(part_ref[...], axis=0, keepdims=True)
    deg = jnp.sum(deg_ref[...], axis=0, keepdims=True)[0:1, 0:1]
    agg = s / jnp.maximum(deg, 1.0)
    e0 = emb_ref[0:1, :]
    h = jnp.maximum(
        jnp.dot(e0, ws_ref[...], preferred_element_type=jnp.float32)
        + jnp.dot(agg, wn_ref[...], preferred_element_type=jnp.float32)
        + bs_ref[...][None, :], 0.0)
    out_ref[...] = (jnp.dot(h, wc_ref[...], preferred_element_type=jnp.float32)
                    + bc_ref[...][None, :])


def kernel(embedding, edges, W_self, W_neigh, b_sage, W_cls, b_cls):
    edges = edges.astype(jnp.int32)

    mesh = plsc.VectorSubcoreMesh(core_axis_name="c", subcore_axis_name="s")
    sc_call = functools.partial(
        pl.kernel,
        mesh=mesh,
        compiler_params=pltpu.CompilerParams(needs_layout_passes=False),
        out_type=(
            jax.ShapeDtypeStruct((NW, D), jnp.float32),
            jax.ShapeDtypeStruct((NW, D), jnp.float32),
        ),
        scratch_types=[
            pltpu.VMEM((WIN,), jnp.int32),
            pltpu.VMEM((WIN,), jnp.int32),
            pltpu.VMEM((LANES,), jnp.int32),
            pltpu.VMEM((LANES, D), jnp.float32),
            pltpu.VMEM((D,), jnp.float32),
            pltpu.VMEM((LANES,), jnp.int32),
            pltpu.VMEM((D,), jnp.float32),
            pltpu.SemaphoreType.DMA,
            pltpu.SemaphoreType.DMA,
            pltpu.SemaphoreType.DMA,
        ],
    )
    partials, degs = sc_call(_sc_filter_gather)(embedding, edges)

    out = pl.pallas_call(
        _tc_finish,
        out_shape=jax.ShapeDtypeStruct((1, OUT), jnp.float32),
        grid=(1,),
        in_specs=[
            pl.BlockSpec((NW, D), lambda i: (0, 0)),
            pl.BlockSpec((NW, D), lambda i: (0, 0)),
            pl.BlockSpec((8, D), lambda i: (0, 0)),
            pl.BlockSpec((D, D), lambda i: (0, 0)),
            pl.BlockSpec((D, D), lambda i: (0, 0)),
            pl.BlockSpec((D,), lambda i: (0,)),
            pl.BlockSpec((D, OUT), lambda i: (0, 0)),
            pl.BlockSpec((OUT,), lambda i: (0,)),
        ],
        out_specs=pl.BlockSpec((1, OUT), lambda i: (0, 0)),
    )(partials, degs, embedding, W_self, W_neigh, b_sage, W_cls, b_cls)

    return out
